# Initial kernel scaffold; baseline (speedup 1.0000x reference)
import jax
import jax.numpy as jnp
from jax.experimental import pallas as pl


def kernel(tokens, wte_weight, wte_override_weight):
    raise NotImplementedError("write your pallas kernel here")


# speedup vs baseline: 1.3232x; 1.3232x over previous
import functools

import jax
import jax.numpy as jnp
from jax import lax
from jax.experimental import pallas as pl
from jax.experimental.pallas import tpu as pltpu
from jax.experimental.pallas import tpu_sc as plsc

_START = 110
_LEN = 800
_NT = 4096 * 50
_NC, _NS, _L = 2, 16, 16
_NW = _NC * _NS
_PER_W = _NT // _NW
_C = 256
_NCHUNK = _PER_W // _C
_GROUPS = _C // _L


@functools.partial(
    pl.kernel,
    out_type=jax.ShapeDtypeStruct((_NT + _L, 128), jnp.float32),
    mesh=plsc.VectorSubcoreMesh(core_axis_name="c", subcore_axis_name="s"),
    compiler_params=pltpu.CompilerParams(needs_layout_passes=False),
    scratch_types=[
        pltpu.VMEM((2, 128), jnp.int32),
        pltpu.VMEM((_C, 128), jnp.float32),
        pltpu.VMEM((_L, 128), jnp.float32),
        pltpu.SemaphoreType.DMA,
    ],
)
def _sc_embed(tok_hbm, wte_hbm, ovr_hbm, out_hbm, idx_v, rows_v, fix_v, sem):
    wid = lax.axis_index("s") * _NC + lax.axis_index("c")
    base = wid * _PER_W
    lanes = lax.iota(jnp.int32, _L)

    def chunk_body(c, carry):
        trow = wid * (_PER_W // 128) + c * (_C // 128)
        pltpu.sync_copy(tok_hbm.at[pl.ds(trow, _C // 128)], idx_v)
        cps = [
            pltpu.async_copy(
                wte_hbm.at[idx_v.at[j]],
                rows_v.at[pl.ds(j * 128, 128)],
                sem,
            )
            for j in range(_C // 128)
        ]
        for cp in cps:
            cp.wait()
        pltpu.sync_copy(rows_v, out_hbm.at[pl.ds(base + c * _C, _C)])

        for g in range(_GROUPS):
            j, col = divmod(g * _L, 128)
            tvec = idx_v[j, pl.ds(col, _L)]
            mask = (tvec >= _START) & (tvec < _START + _LEN)
            nhit = plsc.all_reduce_population_count(mask)

            @pl.when(nhit[0] > 0)
            def _fix(tvec=tvec, mask=mask, g=g, c=c):
                ovr_idx = jnp.where(mask, tvec - _START, 0)
                pos = jnp.where(mask, base + c * _C + g * _L + lanes,
                                _NT + lanes)
                pltpu.async_copy(ovr_hbm.at[ovr_idx], fix_v, sem).wait()
                pltpu.async_copy(fix_v, out_hbm.at[pos], sem).wait()

        return carry

    lax.fori_loop(0, _NCHUNK, chunk_body, 0)


def kernel(tokens, wte_weight, wte_override_weight):
    tok = tokens.astype(jnp.int32).reshape(_NT // 128, 128)
    out = _sc_embed(tok, wte_weight, wte_override_weight)
    return out[:_NT].reshape(4096, 50, 128)


# speedup vs baseline: 4.8777x; 3.6863x over previous
import functools

import jax
import jax.numpy as jnp
from jax import lax
from jax.experimental import pallas as pl
from jax.experimental.pallas import tpu as pltpu
from jax.experimental.pallas import tpu_sc as plsc

_START = 110
_LEN = 800
_NT = 4096 * 50
_NC, _NS, _L = 2, 16, 16
_NW = _NC * _NS
_PER_W = _NT // _NW
_C = 256
_NCHUNK = _PER_W // _C
_GROUPS = _C // _L
_SHIFT = 10


@functools.partial(
    pl.kernel,
    out_type=jax.ShapeDtypeStruct((_NT + _L, 128), jnp.float32),
    mesh=plsc.VectorSubcoreMesh(core_axis_name="c", subcore_axis_name="s"),
    compiler_params=pltpu.CompilerParams(needs_layout_passes=False),
    scratch_types=[
        pltpu.VMEM((_PER_W,), jnp.int32),
        pltpu.VMEM((2, _C, 128), jnp.float32),
        pltpu.VMEM((_PER_W + _L,), jnp.int32),
        pltpu.VMEM((_L, 128), jnp.float32),
        pltpu.SemaphoreType.DMA((2,)),
        pltpu.SemaphoreType.DMA((2,)),
    ],
)
def _sc_embed(tok_hbm, wte_hbm, ovr_hbm, out_hbm, idx_all, rows2, comp_v,
              fix_v, sem_g, sem_w):
    wid = lax.axis_index("s") * _NC + lax.axis_index("c")
    base = wid * _PER_W
    lanes = lax.iota(jnp.int32, _L)

    pltpu.sync_copy(tok_hbm.at[pl.ds(base, _PER_W)], idx_all)

    def start_gather(c):
        p = lax.rem(c, 2)
        for j in range(_C // 128):
            pltpu.async_copy(
                wte_hbm.at[idx_all.at[pl.ds(c * _C + j * 128, 128)]],
                rows2.at[p].at[pl.ds(j * 128, 128)],
                sem_g.at[p],
            )

    def drain_gather(c):
        p = lax.rem(c, 2)
        for j in range(_C // 128):
            pltpu.make_async_copy(
                wte_hbm.at[idx_all.at[pl.ds(c * _C + j * 128, 128)]],
                rows2.at[p].at[pl.ds(j * 128, 128)],
                sem_g.at[p],
            ).wait()

    start_gather(0)
    start_gather(1)

    def chunk_body(c, off):
        p = lax.rem(c, 2)
        drain_gather(c)
        pltpu.async_copy(rows2.at[p], out_hbm.at[pl.ds(base + c * _C, _C)],
                         sem_w.at[p])

        for g in range(_GROUPS):
            tvec = idx_all[pl.ds(c * _C + g * _L, _L)]
            mask = (tvec >= _START) & (tvec < _START + _LEN)
            nhit = plsc.all_reduce_population_count(mask)[0]
            pos = base + c * _C + g * _L + lanes
            combo = (pos << _SHIFT) | (tvec - _START)

            @pl.when(nhit > 0)
            def _store(combo=combo, mask=mask, off=off):
                plsc.store_compressed(comp_v.at[pl.ds(off, _L)],
                                      combo, mask=mask)

            off = off + nhit

        @pl.when(c < _NCHUNK - 2)
        def _next():
            pltpu.make_async_copy(
                rows2.at[p], out_hbm.at[pl.ds(base + c * _C, _C)],
                sem_w.at[p]).wait()
            start_gather(c + 2)

        return off

    n = lax.fori_loop(0, _NCHUNK, chunk_body, 0)

    for c in (_NCHUNK - 2, _NCHUNK - 1):
        p = c % 2
        pltpu.make_async_copy(rows2.at[p],
                              out_hbm.at[pl.ds(base + c * _C, _C)],
                              sem_w.at[p]).wait()

    blk = (n // _L) * _L
    tail = comp_v[pl.ds(blk, _L)]
    pad = (_NT + lanes) << _SHIFT
    comp_v[pl.ds(blk, _L)] = jnp.where(lanes < n - blk, tail, pad)

    def fix_body(b, _):
        vec = comp_v[pl.ds(b * _L, _L)]
        ovr = vec & ((1 << _SHIFT) - 1)
        pos = vec >> _SHIFT
        pltpu.async_copy(ovr_hbm.at[ovr], fix_v, sem_g.at[0]).wait()
        pltpu.async_copy(fix_v, out_hbm.at[pos], sem_g.at[0]).wait()
        return _

    lax.fori_loop(0, (n + _L - 1) // _L, fix_body, 0)


def kernel(tokens, wte_weight, wte_override_weight):
    tok = tokens.astype(jnp.int32).reshape(_NT)
    out = _sc_embed(tok, wte_weight, wte_override_weight)
    return out[:_NT].reshape(4096, 50, 128)
